# baseline (speedup 1.0000x reference)
import math

import jax
import jax.numpy as jnp
from jax import lax
from jax.experimental import pallas as pl
from jax.experimental.pallas import tpu as pltpu
from jax.experimental.pallas import tpu_sc as plsc

F = 128
N_NODES = 10000
N_EDGES = 320000
CUTOFF = 5.0
EPS = 1e-8
NRBF = 20
PAYW = 640

NC = 2
NS = 16
E_CH = 128
COLS = 128
N_GROUPS = PAYW // COLS
CHUNKS = N_EDGES // E_CH
ROWS_PER_SUB = 624
ROWS_TAIL = N_NODES - ROWS_PER_SUB * NS

N_PIPE = 4
E_SLICE = N_EDGES // N_PIPE
CHUNKS_SL = E_SLICE // E_CH
E_BLK = 3200
N_BLK = 1000

_COSC = (0.9999994437335142, -19.739034402899996, 64.9306146958282,
         -85.29598973509471, 58.9126594718367, -21.283218653654167)
_SINC = (6.28318503208348, -41.341616050914666, 81.60091454910972,
         -76.62656447721001, 41.40350492807861, -12.576500880978472)


def _horner(coefs, x):
    acc = jnp.full_like(x, coefs[-1])
    for cf in coefs[-2::-1]:
        acc = acc * x + cf
    return acc


def _sin_2pi(g):
    f = g - jnp.floor(g + 0.5)
    return f * _horner(_SINC, f * f)


def _halfcos_2pi(t):
    f = t - jnp.floor(t + 0.5)
    return 0.5 * _horner(_COSC, f * f) + 0.5


def _pack_mats():
    rr = jnp.arange(8)
    cc40 = jnp.arange(40)
    sum3 = ((cc40[:, None] >= 5 * rr[None, :] + 2)
            & (cc40[:, None] <= 5 * rr[None, :] + 4)).astype(jnp.float32)
    cc160 = jnp.arange(160)
    sel = (cc160[None, :] // NRBF) == rr[:, None]
    rexp = jnp.where(sel, (cc160[None, :] % NRBF + 1).astype(jnp.float32) * 0.1,
                     0.0)
    rone = sel.astype(jnp.float32)
    return sum3, rexp, rone


def _payload_body(r40_ref, Wr_ref, br_ref, sum3_ref, rexp_ref,
                  rone_ref, out_ref):
    EP = E_BLK // 8
    HI = lax.Precision.HIGHEST
    r40 = r40_ref[...]
    n2p = jnp.dot(r40 * r40, sum3_ref[...], preferred_element_type=jnp.float32,
                  precision=HI)
    rawnorm = jnp.sqrt(n2p)
    normcp = jnp.maximum(rawnorm, EPS)
    gp = jnp.dot(normcp, rexp_ref[...], preferred_element_type=jnp.float32,
                 precision=HI)
    invb = jnp.dot(1.0 / (normcp + EPS), rone_ref[...],
                   preferred_element_type=jnp.float32, precision=HI)
    rbfp = jnp.clip(_sin_2pi(gp) * invb, -1.0, 1.0)
    invd = 1.0 / ((rawnorm + EPS) * (rawnorm + EPS))
    for p in range(8):
        rbf = rbfp[:, NRBF * p:NRBF * (p + 1)]
        t = jnp.dot(rbf, Wr_ref[...], preferred_element_type=jnp.float32) + br_ref[...]
        W = jnp.where(t < 0.5, _halfcos_2pi(t), 0.0)
        w3 = W[:, 2 * F:]
        ivr = invd[:, p:p + 1]
        rows = pl.ds(EP * p, EP)
        out_ref[rows, 0:2 * F] = W[:, 0:2 * F]
        out_ref[rows, 2 * F:3 * F] = w3 * (r40[:, 5 * p + 2:5 * p + 3] * ivr)
        out_ref[rows, 3 * F:4 * F] = w3 * (r40[:, 5 * p + 3:5 * p + 4] * ivr)
        out_ref[rows, 4 * F:5 * F] = w3 * (r40[:, 5 * p + 4:5 * p + 5] * ivr)


def _payload_call(r40, Wr, br, packm):
    n_e = r40.shape[0] * 8
    grid = (n_e // E_BLK,)
    sum3, rexp, rone = packm
    return pl.pallas_call(
        _payload_body,
        grid=grid,
        in_specs=[
            pl.BlockSpec((E_BLK // 8, 40), lambda i: (i, 0)),
            pl.BlockSpec((NRBF, 384), lambda i: (0, 0)),
            pl.BlockSpec((1, 384), lambda i: (0, 0)),
            pl.BlockSpec((40, 8), lambda i: (0, 0)),
            pl.BlockSpec((8, 160), lambda i: (0, 0)),
            pl.BlockSpec((8, 160), lambda i: (0, 0)),
        ],
        out_specs=pl.BlockSpec((E_BLK, PAYW), lambda i: (i, 0)),
        out_shape=jax.ShapeDtypeStruct((n_e, PAYW), jnp.float32),
    )(r40, Wr, br, sum3, rexp, rone)


def _sc_scatter_body(payload_hbm, js_hbm, zeros_hbm, acc_hbm,
                     idx_v, buf_v, acc_sh,
                     sem_i0, sem_i1, sem_i2, sem_p0, sem_p1, sem_p2,
                     sem_s0, sem_s1, sem_s2):
    c = lax.axis_index("c")
    s = lax.axis_index("s")
    row0 = pl.multiple_of(s * ROWS_PER_SUB, 8)
    tail0 = ROWS_PER_SUB * NS
    sem_i = (sem_i0, sem_i1, sem_i2)
    sem_p = (sem_p0, sem_p1, sem_p2)
    sem_s = (sem_s0, sem_s1, sem_s2)

    def do_pass(co, out_col, ch0, nch):
        co = pl.multiple_of(co, COLS)
        out_col = pl.multiple_of(out_col, COLS)
        nch_base = nch // NS
        extra = nch - nch_base * NS
        my_n = nch_base + jnp.where(s < extra, 1, 0)

        def _copies(k, b):
            base = pl.multiple_of((ch0 + s + k * NS) * E_CH, E_CH)
            return (
                pltpu.make_async_copy(js_hbm.at[pl.ds(base, E_CH)],
                                      idx_v.at[b], sem_i[b]),
                pltpu.make_async_copy(
                    payload_hbm.at[pl.ds(base, E_CH), pl.ds(co, COLS)],
                    buf_v.at[b], sem_p[b]),
            )

        pltpu.sync_copy(zeros_hbm, acc_sh.at[pl.ds(row0, ROWS_PER_SUB)])

        @pl.when(s == NS - 1)
        def _():
            pltpu.sync_copy(zeros_hbm.at[pl.ds(0, ROWS_TAIL)],
                            acc_sh.at[pl.ds(tail0, ROWS_TAIL)])

        plsc.subcore_barrier()

        def _scat(b):
            return pltpu.make_async_copy(buf_v.at[b],
                                         acc_sh.at[idx_v.at[b]], sem_s[b])

        for b in range(2):
            @pl.when(b < my_n)
            def _(b=b):
                for cp in _copies(b, b):
                    cp.start()

        def outer(k0, carry):
            for b in range(3):
                k = k0 * 3 + b

                @pl.when(k < my_n)
                def _(b=b, k=k):
                    for cp in _copies(k, b):
                        cp.wait()
                    pltpu.async_copy(buf_v.at[b], acc_sh.at[idx_v.at[b]],
                                     sem_s[b], add=True)
                    bp = (b + 2) % 3

                    @pl.when(k >= 1)
                    def __(bp=bp, k=k):
                        _scat(bp).wait()

                    @pl.when(k + 2 < my_n)
                    def __(bp=bp, k=k):
                        for cp in _copies(k + 2, bp):
                            cp.start()
            return carry

        lax.fori_loop(0, (my_n + 2) // 3, outer, 0)
        for b in range(3):
            @pl.when((my_n - 1) % 3 == b)
            def _(b=b):
                _scat(b).wait()
        plsc.subcore_barrier()
        pltpu.sync_copy(acc_sh.at[pl.ds(row0, ROWS_PER_SUB)],
                        acc_hbm.at[pl.ds(row0, ROWS_PER_SUB), pl.ds(out_col, COLS)])

        @pl.when(s == NS - 1)
        def _():
            pltpu.sync_copy(acc_sh.at[pl.ds(tail0, ROWS_TAIL)],
                            acc_hbm.at[pl.ds(tail0, ROWS_TAIL), pl.ds(out_col, COLS)])

    for p in range(2):
        g = 2 * p + c
        do_pass(g * COLS, g * COLS, 0, CHUNKS_SL)
    g4_n0 = CHUNKS_SL // 2 + 1
    do_pass(4 * COLS, 4 * COLS + c * COLS, c * g4_n0, g4_n0 - c)


def _sc_scatter_call(payload, js, zeros_hbm):
    mesh = plsc.VectorSubcoreMesh(core_axis_name="c", subcore_axis_name="s")
    f = pl.kernel(
        _sc_scatter_body,
        out_type=jax.ShapeDtypeStruct((N_NODES, PAYW + COLS), jnp.float32),
        mesh=mesh,
        scratch_types=[
            pltpu.VMEM((3, E_CH), jnp.int32),
            pltpu.VMEM((3, E_CH, COLS), jnp.float32),
            pltpu.VMEM_SHARED((N_NODES, COLS), jnp.float32),
            pltpu.SemaphoreType.DMA,
            pltpu.SemaphoreType.DMA,
            pltpu.SemaphoreType.DMA,
            pltpu.SemaphoreType.DMA,
            pltpu.SemaphoreType.DMA,
            pltpu.SemaphoreType.DMA,
            pltpu.SemaphoreType.DMA,
            pltpu.SemaphoreType.DMA,
            pltpu.SemaphoreType.DMA,
        ],
    )
    return f(payload, js, zeros_hbm)


def _combine_body(s_ref, vt_ref, acc0_ref, acc1_ref, acc2_ref, acc3_ref,
                  W1_ref, b1_ref, W2_ref, b2_ref, dvt_ref, ds_ref):
    h = jnp.dot(s_ref[...], W1_ref[...], preferred_element_type=jnp.float32) + b1_ref[...]
    h = h * jax.nn.sigmoid(h)
    phi = jnp.dot(h, W2_ref[...], preferred_element_type=jnp.float32) + b2_ref[...]

    def a(lo, hi):
        return ((acc0_ref[:, lo:hi] + acc1_ref[:, lo:hi])
                + (acc2_ref[:, lo:hi] + acc3_ref[:, lo:hi]))

    ds_ref[...] = phi[:, F:2 * F] * a(F, 2 * F)
    g1 = phi[:, 0:F] * a(0, F)
    g3 = phi[:, 2 * F:3 * F]
    dvt_ref[:, 0:F] = vt_ref[:, 0:F] * g1 + g3 * a(2 * F, 3 * F)
    dvt_ref[:, F:2 * F] = vt_ref[:, F:2 * F] * g1 + g3 * a(3 * F, 4 * F)
    dvt_ref[:, 2 * F:] = (vt_ref[:, 2 * F:] * g1
                          + g3 * (a(4 * F, 5 * F) + a(5 * F, 6 * F)))


def _combine_call(s, vt, accs, W1, b1, W2, b2):
    grid = (N_NODES // N_BLK,)
    return pl.pallas_call(
        _combine_body,
        grid=grid,
        in_specs=[
            pl.BlockSpec((N_BLK, F), lambda i: (i, 0)),
            pl.BlockSpec((N_BLK, 3 * F), lambda i: (i, 0)),
            pl.BlockSpec((N_BLK, PAYW + COLS), lambda i: (i, 0)),
            pl.BlockSpec((N_BLK, PAYW + COLS), lambda i: (i, 0)),
            pl.BlockSpec((N_BLK, PAYW + COLS), lambda i: (i, 0)),
            pl.BlockSpec((N_BLK, PAYW + COLS), lambda i: (i, 0)),
            pl.BlockSpec((F, F), lambda i: (0, 0)),
            pl.BlockSpec((1, F), lambda i: (0, 0)),
            pl.BlockSpec((F, 3 * F), lambda i: (0, 0)),
            pl.BlockSpec((1, 3 * F), lambda i: (0, 0)),
        ],
        out_specs=[
            pl.BlockSpec((N_BLK, 3 * F), lambda i: (i, 0)),
            pl.BlockSpec((N_BLK, F), lambda i: (i, 0)),
        ],
        out_shape=[
            jax.ShapeDtypeStruct((N_NODES, 3 * F), jnp.float32),
            jax.ShapeDtypeStruct((N_NODES, F), jnp.float32),
        ],
    )(s, vt, *accs, W1, b1, W2, b2)


def kernel(v, s, r, W1, b1, W2, b2, Wr, br):
    js = r[:, 1].astype(jnp.int32)
    zeros_hbm = jnp.zeros((ROWS_PER_SUB, COLS), jnp.float32)
    packm = _pack_mats()
    accs = []
    for hh in range(N_PIPE):
        r_h = lax.slice_in_dim(r, hh * E_SLICE, (hh + 1) * E_SLICE, axis=0)
        r40_h = r_h.reshape(E_SLICE // 8, 40)
        js_h = lax.slice_in_dim(js, hh * E_SLICE, (hh + 1) * E_SLICE, axis=0)
        jsp_h = (js_h.reshape(E_SLICE // E_BLK, E_BLK // 8, 8)
                 .swapaxes(1, 2).reshape(E_SLICE))
        payload_h = _payload_call(r40_h, Wr * (0.5 / CUTOFF),
                                  br.reshape(1, 384) * (0.5 / CUTOFF), packm)
        accs.append(_sc_scatter_call(payload_h, jsp_h, zeros_hbm))
    vt = v.transpose(0, 2, 1).reshape(N_NODES, 3 * F)
    dvt, ds = _combine_call(s, vt, accs, W1, b1.reshape(1, F),
                            W2, b2.reshape(1, 3 * F))
    dv = dvt.reshape(N_NODES, 3, F).transpose(0, 2, 1)
    return dv, ds


# speedup vs baseline: 1.0213x; 1.0213x over previous
import math

import jax
import jax.numpy as jnp
from jax import lax
from jax.experimental import pallas as pl
from jax.experimental.pallas import tpu as pltpu
from jax.experimental.pallas import tpu_sc as plsc

F = 128
N_NODES = 10000
N_EDGES = 320000
CUTOFF = 5.0
EPS = 1e-8
NRBF = 20
PAYW = 640

NC = 2
NS = 16
E_CH = 128
COLS = 128
N_GROUPS = PAYW // COLS
CHUNKS = N_EDGES // E_CH
ROWS_PER_SUB = 624
ROWS_TAIL = N_NODES - ROWS_PER_SUB * NS

N_PIPE = 2
E_SLICE = N_EDGES // N_PIPE
CHUNKS_SL = E_SLICE // E_CH
E_BLK = 6400
N_BLK = 1000

_COSC = (0.9999997218667571, -9.869517201449998, 32.4653073479141,
         -42.647994867547354, 29.45632973591835, -10.641609326827084)
_SINC = (6.28318503208348, -41.341616050914666, 81.60091454910972,
         -76.62656447721001, 41.40350492807861, -12.576500880978472)


def _horner(coefs, x):
    acc = jnp.full_like(x, coefs[-1])
    for cf in coefs[-2::-1]:
        acc = acc * x + cf
    return acc


def _sin_2pi(g):
    f = g - jnp.floor(g + 0.5)
    return f * _horner(_SINC, f * f)


def _halfcos_2pi(t):
    f = t - jnp.floor(t + 0.5)
    return _horner(_COSC, f * f)


def _pack_mats():
    rr = jnp.arange(8)
    cc40 = jnp.arange(40)
    sum3 = ((cc40[:, None] >= 5 * rr[None, :] + 2)
            & (cc40[:, None] <= 5 * rr[None, :] + 4)).astype(jnp.float32)
    cc160 = jnp.arange(160)
    sel = (cc160[None, :] // NRBF) == rr[:, None]
    rexp = jnp.where(sel, (cc160[None, :] % NRBF + 1).astype(jnp.float32) * 0.1,
                     0.0)
    rone = sel.astype(jnp.float32)
    return sum3, rexp, rone


def _payload_body(r40_ref, Wr_ref, br_ref, sum3_ref, rexp_ref,
                  rone_ref, out_ref):
    EP = E_BLK // 8
    HI = lax.Precision.HIGHEST
    r40 = r40_ref[...]
    n2p = jnp.dot(r40 * r40, sum3_ref[...], preferred_element_type=jnp.float32,
                  precision=HI)
    rawnorm = jnp.sqrt(n2p)
    normcp = jnp.maximum(rawnorm, EPS)
    gp = jnp.dot(normcp, rexp_ref[...], preferred_element_type=jnp.float32,
                 precision=HI)
    invb = jnp.dot(1.0 / (normcp + EPS), rone_ref[...],
                   preferred_element_type=jnp.float32, precision=HI)
    rbfp = jnp.clip(_sin_2pi(gp) * invb, -1.0, 1.0)
    invd = 1.0 / ((rawnorm + EPS) * (rawnorm + EPS))
    for p in range(8):
        rbf = rbfp[:, NRBF * p:NRBF * (p + 1)]
        t = jnp.dot(rbf, Wr_ref[...], preferred_element_type=jnp.float32) + br_ref[...]
        W = jnp.where(t < 0.5, _halfcos_2pi(t), 0.0)
        w3 = W[:, 2 * F:]
        ivr = invd[:, p:p + 1]
        rows = pl.ds(EP * p, EP)
        out_ref[rows, 0:2 * F] = W[:, 0:2 * F]
        out_ref[rows, 2 * F:3 * F] = w3 * (r40[:, 5 * p + 2:5 * p + 3] * ivr)
        out_ref[rows, 3 * F:4 * F] = w3 * (r40[:, 5 * p + 3:5 * p + 4] * ivr)
        out_ref[rows, 4 * F:5 * F] = w3 * (r40[:, 5 * p + 4:5 * p + 5] * ivr)


def _payload_call(r40, Wr, br, packm):
    n_e = r40.shape[0] * 8
    grid = (n_e // E_BLK,)
    sum3, rexp, rone = packm
    return pl.pallas_call(
        _payload_body,
        grid=grid,
        in_specs=[
            pl.BlockSpec((E_BLK // 8, 40), lambda i: (i, 0)),
            pl.BlockSpec((NRBF, 384), lambda i: (0, 0)),
            pl.BlockSpec((1, 384), lambda i: (0, 0)),
            pl.BlockSpec((40, 8), lambda i: (0, 0)),
            pl.BlockSpec((8, 160), lambda i: (0, 0)),
            pl.BlockSpec((8, 160), lambda i: (0, 0)),
        ],
        out_specs=pl.BlockSpec((E_BLK, PAYW), lambda i: (i, 0)),
        out_shape=jax.ShapeDtypeStruct((n_e, PAYW), jnp.float32),
    )(r40, Wr, br, sum3, rexp, rone)


def _sc_scatter_body(payload_hbm, js_hbm, zeros_hbm, acc_hbm,
                     idx_v, buf_v, acc_sh,
                     sem_i0, sem_i1, sem_i2, sem_p0, sem_p1, sem_p2,
                     sem_s0, sem_s1, sem_s2):
    c = lax.axis_index("c")
    s = lax.axis_index("s")
    row0 = pl.multiple_of(s * ROWS_PER_SUB, 8)
    tail0 = ROWS_PER_SUB * NS
    sem_i = (sem_i0, sem_i1, sem_i2)
    sem_p = (sem_p0, sem_p1, sem_p2)
    sem_s = (sem_s0, sem_s1, sem_s2)

    def do_pass(co, out_col, ch0, nch):
        co = pl.multiple_of(co, COLS)
        out_col = pl.multiple_of(out_col, COLS)
        nch_base = nch // NS
        extra = nch - nch_base * NS
        my_n = nch_base + jnp.where(s < extra, 1, 0)

        def _copies(k, b):
            base = pl.multiple_of((ch0 + s + k * NS) * E_CH, E_CH)
            return (
                pltpu.make_async_copy(js_hbm.at[pl.ds(base, E_CH)],
                                      idx_v.at[b], sem_i[b]),
                pltpu.make_async_copy(
                    payload_hbm.at[pl.ds(base, E_CH), pl.ds(co, COLS)],
                    buf_v.at[b], sem_p[b]),
            )

        pltpu.sync_copy(zeros_hbm, acc_sh.at[pl.ds(row0, ROWS_PER_SUB)])

        @pl.when(s == NS - 1)
        def _():
            pltpu.sync_copy(zeros_hbm.at[pl.ds(0, ROWS_TAIL)],
                            acc_sh.at[pl.ds(tail0, ROWS_TAIL)])

        plsc.subcore_barrier()

        def _scat(b):
            return pltpu.make_async_copy(buf_v.at[b],
                                         acc_sh.at[idx_v.at[b]], sem_s[b])

        for b in range(2):
            @pl.when(b < my_n)
            def _(b=b):
                for cp in _copies(b, b):
                    cp.start()

        def outer(k0, carry):
            for b in range(3):
                k = k0 * 3 + b

                @pl.when(k < my_n)
                def _(b=b, k=k):
                    for cp in _copies(k, b):
                        cp.wait()
                    pltpu.async_copy(buf_v.at[b], acc_sh.at[idx_v.at[b]],
                                     sem_s[b], add=True)
                    bp = (b + 2) % 3

                    @pl.when(k >= 1)
                    def __(bp=bp, k=k):
                        _scat(bp).wait()

                    @pl.when(k + 2 < my_n)
                    def __(bp=bp, k=k):
                        for cp in _copies(k + 2, bp):
                            cp.start()
            return carry

        lax.fori_loop(0, (my_n + 2) // 3, outer, 0)
        for b in range(3):
            @pl.when((my_n - 1) % 3 == b)
            def _(b=b):
                _scat(b).wait()
        plsc.subcore_barrier()
        pltpu.sync_copy(acc_sh.at[pl.ds(row0, ROWS_PER_SUB)],
                        acc_hbm.at[pl.ds(row0, ROWS_PER_SUB), pl.ds(out_col, COLS)])

        @pl.when(s == NS - 1)
        def _():
            pltpu.sync_copy(acc_sh.at[pl.ds(tail0, ROWS_TAIL)],
                            acc_hbm.at[pl.ds(tail0, ROWS_TAIL), pl.ds(out_col, COLS)])

    for p in range(2):
        g = 2 * p + c
        do_pass(g * COLS, g * COLS, 0, CHUNKS_SL)
    g4_n0 = (CHUNKS_SL + 1) // 2
    do_pass(4 * COLS, 4 * COLS + c * COLS, c * g4_n0,
            g4_n0 - c * (2 * g4_n0 - CHUNKS_SL))


def _sc_scatter_call(payload, js, zeros_hbm):
    mesh = plsc.VectorSubcoreMesh(core_axis_name="c", subcore_axis_name="s")
    f = pl.kernel(
        _sc_scatter_body,
        out_type=jax.ShapeDtypeStruct((N_NODES, PAYW + COLS), jnp.float32),
        mesh=mesh,
        scratch_types=[
            pltpu.VMEM((3, E_CH), jnp.int32),
            pltpu.VMEM((3, E_CH, COLS), jnp.float32),
            pltpu.VMEM_SHARED((N_NODES, COLS), jnp.float32),
            pltpu.SemaphoreType.DMA,
            pltpu.SemaphoreType.DMA,
            pltpu.SemaphoreType.DMA,
            pltpu.SemaphoreType.DMA,
            pltpu.SemaphoreType.DMA,
            pltpu.SemaphoreType.DMA,
            pltpu.SemaphoreType.DMA,
            pltpu.SemaphoreType.DMA,
            pltpu.SemaphoreType.DMA,
        ],
    )
    return f(payload, js, zeros_hbm)


def _combine_body(*refs):
    (s_ref, vt_ref), acc_refs = refs[:2], refs[2:2 + N_PIPE]
    W1_ref, b1_ref, W2_ref, b2_ref, dvt_ref, ds_ref = refs[2 + N_PIPE:]
    h = jnp.dot(s_ref[...], W1_ref[...], preferred_element_type=jnp.float32) + b1_ref[...]
    h = h * jax.nn.sigmoid(h)
    phi = jnp.dot(h, W2_ref[...], preferred_element_type=jnp.float32) + b2_ref[...]

    def a(lo, hi):
        tot = acc_refs[0][:, lo:hi]
        for ar in acc_refs[1:]:
            tot = tot + ar[:, lo:hi]
        return tot

    ds_ref[...] = phi[:, F:2 * F] * a(F, 2 * F)
    g1 = phi[:, 0:F] * a(0, F)
    g3 = phi[:, 2 * F:3 * F]
    dvt_ref[:, 0:F] = vt_ref[:, 0:F] * g1 + g3 * a(2 * F, 3 * F)
    dvt_ref[:, F:2 * F] = vt_ref[:, F:2 * F] * g1 + g3 * a(3 * F, 4 * F)
    dvt_ref[:, 2 * F:] = (vt_ref[:, 2 * F:] * g1
                          + g3 * (a(4 * F, 5 * F) + a(5 * F, 6 * F)))


def _combine_call(s, vt, accs, W1, b1, W2, b2):
    grid = (N_NODES // N_BLK,)
    return pl.pallas_call(
        _combine_body,
        grid=grid,
        in_specs=(
            [pl.BlockSpec((N_BLK, F), lambda i: (i, 0)),
             pl.BlockSpec((N_BLK, 3 * F), lambda i: (i, 0))]
            + [pl.BlockSpec((N_BLK, PAYW + COLS), lambda i: (i, 0))
               for _ in range(N_PIPE)]
            + [pl.BlockSpec((F, F), lambda i: (0, 0)),
               pl.BlockSpec((1, F), lambda i: (0, 0)),
               pl.BlockSpec((F, 3 * F), lambda i: (0, 0)),
               pl.BlockSpec((1, 3 * F), lambda i: (0, 0))]
        ),
        out_specs=[
            pl.BlockSpec((N_BLK, 3 * F), lambda i: (i, 0)),
            pl.BlockSpec((N_BLK, F), lambda i: (i, 0)),
        ],
        out_shape=[
            jax.ShapeDtypeStruct((N_NODES, 3 * F), jnp.float32),
            jax.ShapeDtypeStruct((N_NODES, F), jnp.float32),
        ],
    )(s, vt, *accs, W1, b1, W2, b2)


def kernel(v, s, r, W1, b1, W2, b2, Wr, br):
    js = r[:, 1].astype(jnp.int32)
    zeros_hbm = jnp.zeros((ROWS_PER_SUB, COLS), jnp.float32)
    packm = _pack_mats()
    accs = []
    for hh in range(N_PIPE):
        r_h = lax.slice_in_dim(r, hh * E_SLICE, (hh + 1) * E_SLICE, axis=0)
        r40_h = r_h.reshape(E_SLICE // 8, 40)
        js_h = lax.slice_in_dim(js, hh * E_SLICE, (hh + 1) * E_SLICE, axis=0)
        jsp_h = (js_h.reshape(E_SLICE // E_BLK, E_BLK // 8, 8)
                 .swapaxes(1, 2).reshape(E_SLICE))
        payload_h = _payload_call(r40_h, Wr * (0.5 / CUTOFF),
                                  br.reshape(1, 384) * (0.5 / CUTOFF), packm)
        accs.append(_sc_scatter_call(payload_h, jsp_h, zeros_hbm))
    vt = v.transpose(0, 2, 1).reshape(N_NODES, 3 * F)
    dvt, ds = _combine_call(s, vt, accs, W1, b1.reshape(1, F),
                            W2, b2.reshape(1, 3 * F))
    dv = dvt.reshape(N_NODES, 3, F).transpose(0, 2, 1)
    return dv, ds


# speedup vs baseline: 1.0215x; 1.0002x over previous
import jax
import jax.numpy as jnp
from jax import lax
from jax.experimental import pallas as pl
from jax.experimental.pallas import tpu as pltpu
from jax.experimental.pallas import tpu_sc as plsc

F = 128
N_NODES = 10000
N_EDGES = 320000
CUTOFF = 5.0
EPS = 1e-8
NRBF = 20
PAYW = 640

NC = 2
NS = 16
E_CH = 128
COLS = 128
N_GROUPS = PAYW // COLS
CHUNKS = N_EDGES // E_CH
ROWS_PER_SUB = 624
ROWS_TAIL = N_NODES - ROWS_PER_SUB * NS

N_PIPE = 2
E_SLICE = N_EDGES // N_PIPE
CHUNKS_SL = E_SLICE // E_CH
E_BLK = 6400
N_BLK = 1000

_COSC = (0.9999997218667571, -9.869517201449998, 32.4653073479141,
         -42.647994867547354, 29.45632973591835, -10.641609326827084)
_SINC = (6.28318503208348, -41.341616050914666, 81.60091454910972,
         -76.62656447721001, 41.40350492807861, -12.576500880978472)


def _horner(coefs, x):
    acc = jnp.full_like(x, coefs[-1])
    for cf in coefs[-2::-1]:
        acc = acc * x + cf
    return acc


def _sin_2pi(g):
    f = g - jnp.floor(g + 0.5)
    return f * _horner(_SINC, f * f)


def _halfcos_2pi(t):
    f = t - jnp.floor(t + 0.5)
    return _horner(_COSC, f * f)


def _pack_mats():
    rr = jnp.arange(8)
    cc40 = jnp.arange(40)
    sum3 = ((cc40[:, None] >= 5 * rr[None, :] + 2)
            & (cc40[:, None] <= 5 * rr[None, :] + 4)).astype(jnp.float32)
    cc160 = jnp.arange(160)
    sel = (cc160[None, :] // NRBF) == rr[:, None]
    rexp = jnp.where(sel, (cc160[None, :] % NRBF + 1).astype(jnp.float32) * 0.1,
                     0.0)
    rone = sel.astype(jnp.float32)
    return sum3, rexp, rone


def _payload_body(r40_ref, Wr_ref, br_ref, sum3_ref, rexp_ref,
                  rone_ref, out_ref):
    EP = E_BLK // 8
    HI = lax.Precision.HIGHEST
    r40 = r40_ref[...]
    n2p = jnp.dot(r40 * r40, sum3_ref[...], preferred_element_type=jnp.float32,
                  precision=HI)
    rawnorm = jnp.sqrt(n2p)
    normcp = jnp.maximum(rawnorm, EPS)
    gp = jnp.dot(normcp, rexp_ref[...], preferred_element_type=jnp.float32,
                 precision=HI)
    invb = jnp.dot(1.0 / (normcp + EPS), rone_ref[...],
                   preferred_element_type=jnp.float32, precision=HI)
    rbfp = jnp.clip(_sin_2pi(gp) * invb, -1.0, 1.0)
    invd = 1.0 / ((rawnorm + EPS) * (rawnorm + EPS))
    for p in range(8):
        rbf = rbfp[:, NRBF * p:NRBF * (p + 1)]
        t = jnp.dot(rbf, Wr_ref[...], preferred_element_type=jnp.float32) + br_ref[...]
        W = jnp.where(t < 0.5, _halfcos_2pi(t), 0.0)
        w3 = W[:, 2 * F:]
        ivr = invd[:, p:p + 1]
        rows = pl.ds(EP * p, EP)
        out_ref[rows, 0:2 * F] = W[:, 0:2 * F]
        out_ref[rows, 2 * F:3 * F] = w3 * (r40[:, 5 * p + 2:5 * p + 3] * ivr)
        out_ref[rows, 3 * F:4 * F] = w3 * (r40[:, 5 * p + 3:5 * p + 4] * ivr)
        out_ref[rows, 4 * F:5 * F] = w3 * (r40[:, 5 * p + 4:5 * p + 5] * ivr)


def _payload_call(r40, Wr, br, packm):
    n_e = r40.shape[0] * 8
    grid = (n_e // E_BLK,)
    sum3, rexp, rone = packm
    return pl.pallas_call(
        _payload_body,
        grid=grid,
        in_specs=[
            pl.BlockSpec((E_BLK // 8, 40), lambda i: (i, 0)),
            pl.BlockSpec((NRBF, 384), lambda i: (0, 0)),
            pl.BlockSpec((1, 384), lambda i: (0, 0)),
            pl.BlockSpec((40, 8), lambda i: (0, 0)),
            pl.BlockSpec((8, 160), lambda i: (0, 0)),
            pl.BlockSpec((8, 160), lambda i: (0, 0)),
        ],
        out_specs=pl.BlockSpec((E_BLK, PAYW), lambda i: (i, 0)),
        out_shape=jax.ShapeDtypeStruct((n_e, PAYW), jnp.float32),
    )(r40, Wr, br, sum3, rexp, rone)


def _sc_scatter_body(payload_hbm, js_hbm, zeros_hbm, acc_hbm,
                     idx_v, buf_v, acc_sh,
                     sem_i0, sem_i1, sem_i2, sem_p0, sem_p1, sem_p2,
                     sem_s0, sem_s1, sem_s2):
    c = lax.axis_index("c")
    s = lax.axis_index("s")
    row0 = pl.multiple_of(s * ROWS_PER_SUB, 8)
    tail0 = ROWS_PER_SUB * NS
    sem_i = (sem_i0, sem_i1, sem_i2)
    sem_p = (sem_p0, sem_p1, sem_p2)
    sem_s = (sem_s0, sem_s1, sem_s2)

    def do_pass(co, out_col, ch0, nch):
        co = pl.multiple_of(co, COLS)
        out_col = pl.multiple_of(out_col, COLS)
        nch_base = nch // NS
        extra = nch - nch_base * NS
        my_n = nch_base + jnp.where(s < extra, 1, 0)

        def _copies(k, b):
            base = pl.multiple_of((ch0 + s + k * NS) * E_CH, E_CH)
            return (
                pltpu.make_async_copy(js_hbm.at[pl.ds(base, E_CH)],
                                      idx_v.at[b], sem_i[b]),
                pltpu.make_async_copy(
                    payload_hbm.at[pl.ds(base, E_CH), pl.ds(co, COLS)],
                    buf_v.at[b], sem_p[b]),
            )

        pltpu.sync_copy(zeros_hbm, acc_sh.at[pl.ds(row0, ROWS_PER_SUB)])

        @pl.when(s == NS - 1)
        def _():
            pltpu.sync_copy(zeros_hbm.at[pl.ds(0, ROWS_TAIL)],
                            acc_sh.at[pl.ds(tail0, ROWS_TAIL)])

        plsc.subcore_barrier()

        def _scat(b):
            return pltpu.make_async_copy(buf_v.at[b],
                                         acc_sh.at[idx_v.at[b]], sem_s[b])

        for b in range(2):
            @pl.when(b < my_n)
            def _(b=b):
                for cp in _copies(b, b):
                    cp.start()

        def outer(k0, carry):
            for b in range(3):
                k = k0 * 3 + b

                @pl.when(k < my_n)
                def _(b=b, k=k):
                    for cp in _copies(k, b):
                        cp.wait()
                    pltpu.async_copy(buf_v.at[b], acc_sh.at[idx_v.at[b]],
                                     sem_s[b], add=True)
                    bp = (b + 2) % 3

                    @pl.when(k >= 1)
                    def __(bp=bp, k=k):
                        _scat(bp).wait()

                    @pl.when(k + 2 < my_n)
                    def __(bp=bp, k=k):
                        for cp in _copies(k + 2, bp):
                            cp.start()
            return carry

        lax.fori_loop(0, (my_n + 2) // 3, outer, 0)
        for b in range(3):
            @pl.when((my_n - 1) % 3 == b)
            def _(b=b):
                _scat(b).wait()
        plsc.subcore_barrier()
        pltpu.sync_copy(acc_sh.at[pl.ds(row0, ROWS_PER_SUB)],
                        acc_hbm.at[pl.ds(row0, ROWS_PER_SUB), pl.ds(out_col, COLS)])

        @pl.when(s == NS - 1)
        def _():
            pltpu.sync_copy(acc_sh.at[pl.ds(tail0, ROWS_TAIL)],
                            acc_hbm.at[pl.ds(tail0, ROWS_TAIL), pl.ds(out_col, COLS)])

    for p in range(2):
        g = 2 * p + c
        do_pass(g * COLS, g * COLS, 0, CHUNKS_SL)
    g4_n0 = (CHUNKS_SL + 1) // 2
    do_pass(4 * COLS, 4 * COLS + c * COLS, c * g4_n0,
            g4_n0 - c * (2 * g4_n0 - CHUNKS_SL))


def _sc_scatter_call(payload, js, zeros_hbm):
    mesh = plsc.VectorSubcoreMesh(core_axis_name="c", subcore_axis_name="s")
    f = pl.kernel(
        _sc_scatter_body,
        out_type=jax.ShapeDtypeStruct((N_NODES, PAYW + COLS), jnp.float32),
        mesh=mesh,
        scratch_types=[
            pltpu.VMEM((3, E_CH), jnp.int32),
            pltpu.VMEM((3, E_CH, COLS), jnp.float32),
            pltpu.VMEM_SHARED((N_NODES, COLS), jnp.float32),
            pltpu.SemaphoreType.DMA,
            pltpu.SemaphoreType.DMA,
            pltpu.SemaphoreType.DMA,
            pltpu.SemaphoreType.DMA,
            pltpu.SemaphoreType.DMA,
            pltpu.SemaphoreType.DMA,
            pltpu.SemaphoreType.DMA,
            pltpu.SemaphoreType.DMA,
            pltpu.SemaphoreType.DMA,
        ],
    )
    return f(payload, js, zeros_hbm)


def _combine_body(*refs):
    (s_ref, vt_ref), acc_refs = refs[:2], refs[2:2 + N_PIPE]
    W1_ref, b1_ref, W2_ref, b2_ref, dvt_ref, ds_ref = refs[2 + N_PIPE:]
    h = jnp.dot(s_ref[...], W1_ref[...], preferred_element_type=jnp.float32) + b1_ref[...]
    h = h * jax.nn.sigmoid(h)
    phi = jnp.dot(h, W2_ref[...], preferred_element_type=jnp.float32) + b2_ref[...]

    def a(lo, hi):
        tot = acc_refs[0][:, lo:hi]
        for ar in acc_refs[1:]:
            tot = tot + ar[:, lo:hi]
        return tot

    ds_ref[...] = phi[:, F:2 * F] * a(F, 2 * F)
    g1 = phi[:, 0:F] * a(0, F)
    g3 = phi[:, 2 * F:3 * F]
    dvt_ref[:, 0:F] = vt_ref[:, 0:F] * g1 + g3 * a(2 * F, 3 * F)
    dvt_ref[:, F:2 * F] = vt_ref[:, F:2 * F] * g1 + g3 * a(3 * F, 4 * F)
    dvt_ref[:, 2 * F:] = (vt_ref[:, 2 * F:] * g1
                          + g3 * (a(4 * F, 5 * F) + a(5 * F, 6 * F)))


def _combine_call(s, vt, accs, W1, b1, W2, b2):
    grid = (N_NODES // N_BLK,)
    return pl.pallas_call(
        _combine_body,
        grid=grid,
        in_specs=(
            [pl.BlockSpec((N_BLK, F), lambda i: (i, 0)),
             pl.BlockSpec((N_BLK, 3 * F), lambda i: (i, 0))]
            + [pl.BlockSpec((N_BLK, PAYW + COLS), lambda i: (i, 0))
               for _ in range(N_PIPE)]
            + [pl.BlockSpec((F, F), lambda i: (0, 0)),
               pl.BlockSpec((1, F), lambda i: (0, 0)),
               pl.BlockSpec((F, 3 * F), lambda i: (0, 0)),
               pl.BlockSpec((1, 3 * F), lambda i: (0, 0))]
        ),
        out_specs=[
            pl.BlockSpec((N_BLK, 3 * F), lambda i: (i, 0)),
            pl.BlockSpec((N_BLK, F), lambda i: (i, 0)),
        ],
        out_shape=[
            jax.ShapeDtypeStruct((N_NODES, 3 * F), jnp.float32),
            jax.ShapeDtypeStruct((N_NODES, F), jnp.float32),
        ],
    )(s, vt, *accs, W1, b1, W2, b2)


def kernel(v, s, r, W1, b1, W2, b2, Wr, br):
    js = r[:, 1].astype(jnp.int32)
    zeros_hbm = jnp.zeros((ROWS_PER_SUB, COLS), jnp.float32)
    packm = _pack_mats()
    accs = []
    for hh in range(N_PIPE):
        r_h = lax.slice_in_dim(r, hh * E_SLICE, (hh + 1) * E_SLICE, axis=0)
        r40_h = r_h.reshape(E_SLICE // 8, 40)
        js_h = lax.slice_in_dim(js, hh * E_SLICE, (hh + 1) * E_SLICE, axis=0)
        jsp_h = (js_h.reshape(E_SLICE // E_BLK, E_BLK // 8, 8)
                 .swapaxes(1, 2).reshape(E_SLICE))
        payload_h = _payload_call(r40_h, Wr * (0.5 / CUTOFF),
                                  br.reshape(1, 384) * (0.5 / CUTOFF), packm)
        accs.append(_sc_scatter_call(payload_h, jsp_h, zeros_hbm))
    vt = v.transpose(0, 2, 1).reshape(N_NODES, 3 * F)
    dvt, ds = _combine_call(s, vt, accs, W1, b1.reshape(1, F),
                            W2, b2.reshape(1, 3 * F))
    dv = dvt.reshape(N_NODES, 3, F).transpose(0, 2, 1)
    return dv, ds
